# baseline (speedup 1.0000x reference)
import functools

import jax
import jax.numpy as jnp
from jax import lax
from jax.experimental import pallas as pl
from jax.experimental.pallas import tpu as pltpu
from jax.experimental.pallas import tpu_sc as plsc

EMB = 64
NUM_WORKERS = 32
CHUNK = 640


@functools.partial(jax.jit, static_argnums=(2,))
def _sc_gather(flat_idx, table, n_rows):
    b_per_w = n_rows // NUM_WORKERS
    n_chunks = b_per_w // CHUNK
    mesh = plsc.VectorSubcoreMesh(core_axis_name="c", subcore_axis_name="s")

    @functools.partial(
        pl.kernel,
        mesh=mesh,
        out_type=jax.ShapeDtypeStruct((n_rows, EMB), jnp.float32),
        scratch_types=[
            pltpu.VMEM((CHUNK,), jnp.int32),
            pltpu.VMEM((CHUNK, EMB), jnp.float32),
            pltpu.SemaphoreType.DMA,
        ],
        compiler_params=pltpu.CompilerParams(use_tc_tiling_on_sc=False),
    )
    def k(idx_hbm, table_hbm, out_hbm, idx_v, rows_v, sem):
        wid = lax.axis_index("s") * 2 + lax.axis_index("c")
        base = wid * b_per_w

        def body(i, carry):
            off = base + i * CHUNK
            pltpu.sync_copy(idx_hbm.at[pl.ds(off, CHUNK)], idx_v)
            pltpu.async_copy(table_hbm.at[idx_v], rows_v, sem).wait()
            pltpu.sync_copy(rows_v, out_hbm.at[pl.ds(off, CHUNK)])
            return carry

        lax.fori_loop(0, n_chunks, body, 0)

    return k(flat_idx, table)


def kernel(inputs, table):
    b, s = inputs.shape
    flat_idx = inputs.reshape(-1).astype(jnp.int32)
    out = _sc_gather(flat_idx, table, b * s)
    return out.reshape(b, s, EMB)


# speedup vs baseline: 1.0099x; 1.0099x over previous
import functools

import jax
import jax.numpy as jnp
from jax import lax
from jax.experimental import pallas as pl
from jax.experimental.pallas import tpu as pltpu
from jax.experimental.pallas import tpu_sc as plsc

EMB = 64
NUM_WORKERS = 32
CHUNK = 640


NBUF = 2


@functools.partial(jax.jit, static_argnums=(2,))
def _sc_gather(flat_idx, table, n_rows):
    b_per_w = n_rows // NUM_WORKERS
    n_chunks = b_per_w // CHUNK
    mesh = plsc.VectorSubcoreMesh(core_axis_name="c", subcore_axis_name="s")

    scratch = []
    for _ in range(NBUF):
        scratch += [
            pltpu.VMEM((CHUNK,), jnp.int32),
            pltpu.VMEM((CHUNK, EMB), jnp.float32),
            pltpu.SemaphoreType.DMA,
            pltpu.SemaphoreType.DMA,
        ]

    @functools.partial(
        pl.kernel,
        mesh=mesh,
        out_type=jax.ShapeDtypeStruct((n_rows, EMB), jnp.float32),
        scratch_types=scratch,
        compiler_params=pltpu.CompilerParams(use_tc_tiling_on_sc=False),
    )
    def k(idx_hbm, table_hbm, out_hbm, *bufs):
        wid = lax.axis_index("s") * 2 + lax.axis_index("c")
        base = wid * b_per_w
        slots = [tuple(bufs[4 * b : 4 * b + 4]) for b in range(NBUF)]
        gather_cp = [None] * n_chunks
        write_cp = [None] * n_chunks

        def start(i):
            ib, rb, gsm, _ = slots[i % NBUF]
            if i >= NBUF:
                write_cp[i - NBUF].wait()
            pltpu.sync_copy(idx_hbm.at[pl.ds(base + i * CHUNK, CHUNK)], ib)
            gather_cp[i] = pltpu.async_copy(table_hbm.at[ib], rb, gsm)

        for i in range(min(NBUF, n_chunks)):
            start(i)
        for i in range(n_chunks):
            gather_cp[i].wait()
            _, rb, _, wsm = slots[i % NBUF]
            write_cp[i] = pltpu.async_copy(
                rb, out_hbm.at[pl.ds(base + i * CHUNK, CHUNK)], wsm)
            if i + NBUF < n_chunks:
                start(i + NBUF)
        for i in range(max(0, n_chunks - NBUF), n_chunks):
            write_cp[i].wait()

    return k(flat_idx, table)


def kernel(inputs, table):
    b, s = inputs.shape
    flat_idx = inputs.reshape(-1).astype(jnp.int32)
    out = _sc_gather(flat_idx, table, b * s)
    return out.reshape(b, s, EMB)
